# Initial kernel scaffold; baseline (speedup 1.0000x reference)
import jax
import jax.numpy as jnp
from jax.experimental import pallas as pl


def kernel(in_feat, coord_feat, edge_index, We1, be1, We2, be2, Wc1, bc1, Wc2, Wn1, bn1, Wn2, bn2, Wlh, blh):
    raise NotImplementedError("write your pallas kernel here")


# speedup vs baseline: 2.4452x; 2.4452x over previous
import functools

import jax
import jax.numpy as jnp
from jax import lax
from jax.experimental import pallas as pl
from jax.experimental.pallas import tpu as pltpu
from jax.experimental.pallas import tpu_sc as plsc

N = 10000
E = 320000
D = 128
C = 40
LAYERS = 5
W = 144
XW = W - D
NC, NS = 2, 16
NW = NC * NS
BC = 128
NCHUNK = E // BC
ROWS_PER_SUB = N // NS
BE = 2000
BN = 1000

_mesh = plsc.VectorSubcoreMesh(core_axis_name="c", subcore_axis_name="s")


def _silu(v):
    return v / (1.0 + jnp.exp(-v))


def _leaky(v):
    return jnp.where(v >= 0, v, 0.01 * v)


def _dot(a, b):
    return jnp.dot(a, b, preferred_element_type=jnp.float32)


@functools.partial(
    pl.kernel,
    out_type=jax.ShapeDtypeStruct((E, W), jnp.float32),
    mesh=_mesh,
    scratch_types=[
        pltpu.VMEM((BC,), jnp.int32),
        pltpu.VMEM((BC,), jnp.int32),
        pltpu.VMEM((BC, W), jnp.float32),
        pltpu.VMEM((BC, W), jnp.float32),
        pltpu.SemaphoreType.DMA,
        pltpu.SemaphoreType.DMA,
    ],
    compiler_params=pltpu.CompilerParams(use_tc_tiling_on_sc=False),
)
def _gather(pa, pb, src, dst, out, idxs, idxd, r1, r2, sem1, sem2):
    cid = lax.axis_index("c")
    sid = lax.axis_index("s")
    wid = sid * NC + cid
    nmine = (NCHUNK - wid + NW - 1) // NW

    def chunk(k, _):
        base = (wid + k * NW) * BC
        pltpu.sync_copy(src.at[pl.ds(base, BC)], idxs)
        pltpu.sync_copy(dst.at[pl.ds(base, BC)], idxd)
        c1 = pltpu.async_copy(pa.at[idxs], r1, sem1)
        c2 = pltpu.async_copy(pb.at[idxd], r2, sem2)
        c1.wait()
        c2.wait()

        def arow(r, carry):
            for kk in range(W // 16):
                sl = pl.ds(kk * 16, 16)
                r1[r, sl] = r1[r, sl] + r2[r, sl]
            return carry

        lax.fori_loop(0, BC, arow, 0)
        pltpu.sync_copy(r1, out.at[pl.ds(base, BC)])
        return _

    lax.fori_loop(0, nmine, chunk, 0)


@functools.partial(
    pl.kernel,
    out_type=jax.ShapeDtypeStruct((NC, N, W), jnp.float32),
    mesh=_mesh,
    scratch_types=[
        pltpu.VMEM((BC,), jnp.int32),
        pltpu.VMEM((BC, W), jnp.float32),
        pltpu.VMEM((BC, W), jnp.float32),
        pltpu.VMEM_SHARED((N, W), jnp.float32),
    ],
    compiler_params=pltpu.CompilerParams(use_tc_tiling_on_sc=False),
)
def _scatter(msg, dst, out, idxd, mbuf, zbuf, acc):
    cid = lax.axis_index("c")
    sid = lax.axis_index("s")
    wid = sid * NC + cid

    def zrow(r, carry):
        for kk in range(W // 16):
            zbuf[r, pl.ds(kk * 16, 16)] = jnp.zeros((16,), jnp.float32)
        return carry

    lax.fori_loop(0, BC, zrow, 0)

    row0 = sid * ROWS_PER_SUB
    nfull = ROWS_PER_SUB // BC
    rem = ROWS_PER_SUB - nfull * BC
    for k in range(nfull):
        pltpu.sync_copy(zbuf, acc.at[pl.ds(row0 + k * BC, BC)])
    pltpu.sync_copy(zbuf.at[pl.ds(0, rem)], acc.at[pl.ds(row0 + nfull * BC, rem)])
    plsc.subcore_barrier()

    nmine = (NCHUNK - wid + NW - 1) // NW

    def chunk(k, _):
        base = (wid + k * NW) * BC
        pltpu.sync_copy(dst.at[pl.ds(base, BC)], idxd)
        pltpu.sync_copy(msg.at[pl.ds(base, BC)], mbuf)
        pltpu.sync_copy(mbuf, acc.at[idxd], add=True)
        return _

    lax.fori_loop(0, nmine, chunk, 0)
    plsc.subcore_barrier()

    for k in range(nfull):
        pltpu.sync_copy(acc.at[pl.ds(row0 + k * BC, BC)], zbuf)
        pltpu.sync_copy(zbuf, out.at[cid, pl.ds(row0 + k * BC, BC)])
    pltpu.sync_copy(acc.at[pl.ds(row0 + nfull * BC, rem)], zbuf.at[pl.ds(0, rem)])
    pltpu.sync_copy(zbuf.at[pl.ds(0, rem)], out.at[cid, pl.ds(row0 + nfull * BC, rem)])


def _prep_body(h_ref, x_ref, wa_ref, wb_ref, ba_ref, pa_ref, pb_ref):
    h = h_ref[...]
    x = x_ref[...]
    pa_ref[:, :D] = _dot(h, wa_ref[...]) + ba_ref[...]
    pa_ref[:, D:W] = x
    pb_ref[:, :D] = _dot(h, wb_ref[...])
    pb_ref[:, D:W] = -x


def _wspec(shape):
    return pl.BlockSpec(shape, lambda i: (0, 0))


_prep = pl.pallas_call(
    _prep_body,
    grid=(N // BN,),
    in_specs=[
        pl.BlockSpec((BN, D), lambda i: (i, 0)),
        pl.BlockSpec((BN, XW), lambda i: (i, 0)),
        _wspec((D, D)),
        _wspec((D, D)),
        _wspec((1, D)),
    ],
    out_specs=[
        pl.BlockSpec((BN, W), lambda i: (i, 0)),
        pl.BlockSpec((BN, W), lambda i: (i, 0)),
    ],
    out_shape=[
        jax.ShapeDtypeStruct((N, W), jnp.float32),
        jax.ShapeDtypeStruct((N, W), jnp.float32),
    ],
)


def _edge_body(ein_ref, wr_ref, w2_ref, b2_ref, wc1_ref, bc1_ref, wc2_ref, out_ref):
    e = ein_ref[:, :D]
    tail = ein_ref[:, D:W]
    radial = jnp.sum(tail * tail, axis=1, keepdims=True)
    m = _silu(e + radial * wr_ref[...])
    msg = _silu(_dot(m, w2_ref[...]) + b2_ref[...])
    cc = _silu(_dot(msg, wc1_ref[...]) + bc1_ref[...])
    s = _dot(cc, wc2_ref[...])
    scale = s / (jnp.sqrt(radial) + 1e-30)
    lane = lax.broadcasted_iota(jnp.int32, (BE, XW), 1)
    out_ref[:, :D] = msg
    out_ref[:, D:W] = jnp.where(lane == 3, 1.0, tail * scale)


_edge = pl.pallas_call(
    _edge_body,
    grid=(E // BE,),
    in_specs=[
        pl.BlockSpec((BE, W), lambda i: (i, 0)),
        _wspec((1, D)),
        _wspec((D, D)),
        _wspec((1, D)),
        _wspec((D, D)),
        _wspec((1, D)),
        _wspec((D, 1)),
    ],
    out_specs=pl.BlockSpec((BE, W), lambda i: (i, 0)),
    out_shape=jax.ShapeDtypeStruct((E, W), jnp.float32),
)


def _node_common(h_ref, x_ref, part_ref, wn1a, wn1b, bn1, wn2, bn2):
    hn = part_ref[0, :, :D] + part_ref[1, :, :D]
    tail = part_ref[0, :, D:W] + part_ref[1, :, D:W]
    deg = jnp.maximum(tail[:, 3:4], 1.0)
    lane = lax.broadcasted_iota(jnp.int32, (BN, XW), 1)
    xn = jnp.where(lane < 3, tail / deg, 0.0)
    h = h_ref[...]
    t = _silu(_dot(h, wn1a[...]) + _dot(hn, wn1b[...]) + bn1[...])
    hq = _dot(t, wn2[...]) + bn2[...]
    hnew = _leaky(hq)
    xnew = _leaky(x_ref[...] + xn)
    return hnew, xnew


def _node_body(h_ref, x_ref, part_ref, wn1a, wn1b, bn1, wn2, bn2, wa, wb, ba,
               h_out, x_out, pa_out, pb_out):
    hnew, xnew = _node_common(h_ref, x_ref, part_ref, wn1a, wn1b, bn1, wn2, bn2)
    h_out[...] = hnew
    x_out[...] = xnew
    pa_out[:, :D] = _dot(hnew, wa[...]) + ba[...]
    pa_out[:, D:W] = xnew
    pb_out[:, :D] = _dot(hnew, wb[...])
    pb_out[:, D:W] = -xnew


def _node_last_body(h_ref, x_ref, part_ref, wn1a, wn1b, bn1, wn2, bn2, wlh, blh,
                    out_ref):
    hnew, _ = _node_common(h_ref, x_ref, part_ref, wn1a, wn1b, bn1, wn2, bn2)
    out_ref[...] = _dot(hnew, wlh[...]) + blh[...]


_node_in_specs = [
    pl.BlockSpec((BN, D), lambda i: (i, 0)),
    pl.BlockSpec((BN, XW), lambda i: (i, 0)),
    pl.BlockSpec((NC, BN, W), lambda i: (0, i, 0)),
    _wspec((D, D)),
    _wspec((D, D)),
    _wspec((1, D)),
    _wspec((D, D)),
    _wspec((1, D)),
]

_node = pl.pallas_call(
    _node_body,
    grid=(N // BN,),
    in_specs=_node_in_specs + [_wspec((D, D)), _wspec((D, D)), _wspec((1, D))],
    out_specs=[
        pl.BlockSpec((BN, D), lambda i: (i, 0)),
        pl.BlockSpec((BN, XW), lambda i: (i, 0)),
        pl.BlockSpec((BN, W), lambda i: (i, 0)),
        pl.BlockSpec((BN, W), lambda i: (i, 0)),
    ],
    out_shape=[
        jax.ShapeDtypeStruct((N, D), jnp.float32),
        jax.ShapeDtypeStruct((N, XW), jnp.float32),
        jax.ShapeDtypeStruct((N, W), jnp.float32),
        jax.ShapeDtypeStruct((N, W), jnp.float32),
    ],
)

_node_last = pl.pallas_call(
    _node_last_body,
    grid=(N // BN,),
    in_specs=_node_in_specs + [_wspec((D, C)), _wspec((1, C))],
    out_specs=pl.BlockSpec((BN, C), lambda i: (i, 0)),
    out_shape=jax.ShapeDtypeStruct((N, C), jnp.float32),
)


def kernel(in_feat, coord_feat, edge_index, We1, be1, We2, be2, Wc1, bc1, Wc2,
           Wn1, bn1, Wn2, bn2, Wlh, blh):
    src = edge_index[0]
    dst = edge_index[1]
    x = jnp.concatenate([coord_feat, jnp.zeros((N, XW - 3), jnp.float32)], axis=1)
    h = in_feat
    pa, pb = _prep(h, x, We1[0, :D], We1[0, D:2 * D], be1[0][None])
    out = None
    for i in range(LAYERS):
        ein = _gather(pa, pb, src, dst)
        msg = _edge(ein, We1[i, 2 * D][None], We2[i], be2[i][None],
                    Wc1[i], bc1[i][None], Wc2[i])
        part = _scatter(msg, dst)
        if i < LAYERS - 1:
            h, x, pa, pb = _node(h, x, part, Wn1[i, :D], Wn1[i, D:],
                                 bn1[i][None], Wn2[i], bn2[i][None],
                                 We1[i + 1, :D], We1[i + 1, D:2 * D],
                                 be1[i + 1][None])
        else:
            out = _node_last(h, x, part, Wn1[i, :D], Wn1[i, D:],
                             bn1[i][None], Wn2[i], bn2[i][None], Wlh, blh[None])
    return out


# speedup vs baseline: 2.6896x; 1.1000x over previous
import functools

import jax
import jax.numpy as jnp
from jax import lax
from jax.experimental import pallas as pl
from jax.experimental.pallas import tpu as pltpu
from jax.experimental.pallas import tpu_sc as plsc

N = 10000
E = 320000
D = 128
C = 40
LAYERS = 5
W = 144
XW = W - D
NC, NS = 2, 16
NW = NC * NS
BC = 128
NCHUNK = E // BC
CPS = NCHUNK // NW
XCH = NCHUNK - NW * CPS
ROWS_PER_SUB = N // NS
BE = 2000
BN = 1000

_mesh = plsc.VectorSubcoreMesh(core_axis_name="c", subcore_axis_name="s")


def _silu(v):
    return v / (1.0 + jnp.exp(-v))


def _leaky(v):
    return jnp.where(v >= 0, v, 0.01 * v)


def _dot(a, b):
    return jnp.dot(a, b, preferred_element_type=jnp.float32)


@functools.partial(
    pl.kernel,
    out_type=jax.ShapeDtypeStruct((E, W), jnp.float32),
    mesh=_mesh,
    scratch_types=[
        pltpu.VMEM((CPS + 1, BC), jnp.int32),
        pltpu.VMEM((CPS + 1, BC), jnp.int32),
        pltpu.VMEM((2 * BC, W), jnp.float32),
        pltpu.VMEM((2 * BC, W), jnp.float32),
        pltpu.VMEM((BC, W), jnp.float32),
        pltpu.SemaphoreType.DMA,
        pltpu.SemaphoreType.DMA,
        pltpu.SemaphoreType.DMA,
    ],
    compiler_params=pltpu.CompilerParams(use_tc_tiling_on_sc=False),
)
def _gather(tab, src_i, dst_i, out, idxs, idxd, r0, r1, sbuf, gsem0, gsem1, ssem):
    cid = lax.axis_index("c")
    sid = lax.axis_index("s")
    wid = sid * NC + cid
    c0 = wid * CPS

    pltpu.sync_copy(src_i.at[pl.ds(c0, CPS)], idxs.at[pl.ds(0, CPS)])
    pltpu.sync_copy(dst_i.at[pl.ds(c0, CPS)], idxd.at[pl.ds(0, CPS)])

    @pl.when(wid < XCH)
    def _():
        pltpu.sync_copy(src_i.at[pl.ds(NW * CPS + wid, 1)], idxs.at[pl.ds(CPS, 1)])
        pltpu.sync_copy(dst_i.at[pl.ds(NW * CPS + wid, 1)], idxd.at[pl.ds(CPS, 1)])

    def gchunk(k):
        return jnp.where(k < CPS, c0 + k, NW * CPS + wid)

    def issue_g(k, rbuf, gsem):
        pltpu.async_copy(tab.at[idxs.at[k]], rbuf.at[pl.ds(0, BC)], gsem)
        pltpu.async_copy(tab.at[idxd.at[k]], rbuf.at[pl.ds(BC, BC)], gsem)

    def wait_g(rbuf, gsem):
        pltpu.make_async_copy(tab.at[pl.ds(0, 2 * BC)], rbuf, gsem).wait()

    def wait_s():
        pltpu.make_async_copy(tab.at[pl.ds(0, BC)], sbuf, ssem).wait()

    def add_store(k, rbuf):
        @functools.partial(plsc.parallel_loop, 0, BC, unroll=4)
        def _(r):
            for kk in range(W // 16):
                sl = pl.ds(kk * 16, 16)
                sbuf[r, sl] = rbuf[r, sl] + rbuf[r + BC, sl]

        pltpu.async_copy(sbuf, out.at[pl.ds(gchunk(k) * BC, BC)], ssem)

    _ = (r1, gsem1)
    nmine = CPS + jnp.where(wid < XCH, 1, 0)

    def chunk(k, carry):
        issue_g(k, r0, gsem0)
        wait_g(r0, gsem0)
        add_store(k, r0)
        wait_s()
        return carry

    lax.fori_loop(0, nmine, chunk, 0)


@functools.partial(
    pl.kernel,
    out_type=jax.ShapeDtypeStruct((NC, N, W), jnp.float32),
    mesh=_mesh,
    scratch_types=[
        pltpu.VMEM((1, BC), jnp.int32),
        pltpu.VMEM((1, BC), jnp.int32),
        pltpu.VMEM((BC, W), jnp.float32),
        pltpu.VMEM((BC, W), jnp.float32),
        pltpu.VMEM_SHARED((N, W), jnp.float32),
        pltpu.SemaphoreType.DMA,
        pltpu.SemaphoreType.DMA,
        pltpu.SemaphoreType.DMA,
        pltpu.SemaphoreType.DMA,
    ],
    compiler_params=pltpu.CompilerParams(use_tc_tiling_on_sc=False),
)
def _scatter(msg, dst_i, zeros, out, i0, i1, m0, m1, acc, ms0, ms1, as0, as1):
    cid = lax.axis_index("c")
    sid = lax.axis_index("s")
    wid = sid * NC + cid
    c0 = wid * CPS

    _ = (i1, m1, ms0, ms1, as0, as1)
    row0 = sid * ROWS_PER_SUB
    nfull = ROWS_PER_SUB // BC
    rem = ROWS_PER_SUB - nfull * BC
    for k in range(nfull):
        pltpu.sync_copy(zeros, acc.at[pl.ds(row0 + k * BC, BC)])
    pltpu.sync_copy(zeros.at[pl.ds(0, rem)], acc.at[pl.ds(row0 + nfull * BC, rem)])
    plsc.subcore_barrier()

    def gchunk(k):
        return jnp.where(k < CPS, c0 + k, NW * CPS + wid)

    nmine = CPS + jnp.where(wid < XCH, 1, 0)

    def chunk(k, carry):
        gc = gchunk(k)
        pltpu.sync_copy(dst_i.at[pl.ds(gc, 1)], i0)
        pltpu.sync_copy(msg.at[pl.ds(gc * BC, BC)], m0)
        pltpu.sync_copy(m0, acc.at[i0.at[0]], add=True)
        return carry

    lax.fori_loop(0, nmine, chunk, 0)
    plsc.subcore_barrier()

    for k in range(nfull):
        pltpu.sync_copy(acc.at[pl.ds(row0 + k * BC, BC)],
                        out.at[cid, pl.ds(row0 + k * BC, BC)])
    pltpu.sync_copy(acc.at[pl.ds(row0 + nfull * BC, rem)],
                    out.at[cid, pl.ds(row0 + nfull * BC, rem)])


def _prep_body(h_ref, x_ref, wa_ref, wb_ref, ba_ref, t_ref):
    h = h_ref[...]
    x = x_ref[...]
    t_ref[0, :, :D] = _dot(h, wa_ref[...]) + ba_ref[...]
    t_ref[0, :, D:W] = x
    t_ref[1, :, :D] = _dot(h, wb_ref[...])
    t_ref[1, :, D:W] = -x


def _wspec(shape):
    return pl.BlockSpec(shape, lambda i: (0, 0))


_prep = pl.pallas_call(
    _prep_body,
    grid=(N // BN,),
    in_specs=[
        pl.BlockSpec((BN, D), lambda i: (i, 0)),
        pl.BlockSpec((BN, XW), lambda i: (i, 0)),
        _wspec((D, D)),
        _wspec((D, D)),
        _wspec((1, D)),
    ],
    out_specs=pl.BlockSpec((2, BN, W), lambda i: (0, i, 0)),
    out_shape=jax.ShapeDtypeStruct((2, N, W), jnp.float32),
)


def _edge_body(ein_ref, wr_ref, w2_ref, b2_ref, wc1_ref, bc1_ref, wc2_ref, out_ref):
    e = ein_ref[:, :D]
    tail = ein_ref[:, D:W]
    radial = jnp.sum(tail * tail, axis=1, keepdims=True)
    m = _silu(e + radial * wr_ref[...])
    msg = _silu(_dot(m, w2_ref[...]) + b2_ref[...])
    cc = _silu(_dot(msg, wc1_ref[...]) + bc1_ref[...])
    s = _dot(cc, wc2_ref[...])
    scale = s / (jnp.sqrt(radial) + 1e-30)
    lane = lax.broadcasted_iota(jnp.int32, (BE, XW), 1)
    out_ref[:, :D] = msg
    out_ref[:, D:W] = jnp.where(lane == 3, 1.0, tail * scale)


_edge = pl.pallas_call(
    _edge_body,
    grid=(E // BE,),
    in_specs=[
        pl.BlockSpec((BE, W), lambda i: (i, 0)),
        _wspec((1, D)),
        _wspec((D, D)),
        _wspec((1, D)),
        _wspec((D, D)),
        _wspec((1, D)),
        _wspec((D, 1)),
    ],
    out_specs=pl.BlockSpec((BE, W), lambda i: (i, 0)),
    out_shape=jax.ShapeDtypeStruct((E, W), jnp.float32),
)


def _node_common(h_ref, x_ref, part_ref, wn1a, wn1b, bn1, wn2, bn2):
    hn = part_ref[0, :, :D] + part_ref[1, :, :D]
    tail = part_ref[0, :, D:W] + part_ref[1, :, D:W]
    deg = jnp.maximum(tail[:, 3:4], 1.0)
    lane = lax.broadcasted_iota(jnp.int32, (BN, XW), 1)
    xn = jnp.where(lane < 3, tail / deg, 0.0)
    h = h_ref[...]
    t = _silu(_dot(h, wn1a[...]) + _dot(hn, wn1b[...]) + bn1[...])
    hq = _dot(t, wn2[...]) + bn2[...]
    hnew = _leaky(hq)
    xnew = _leaky(x_ref[...] + xn)
    return hnew, xnew


def _node_body(h_ref, x_ref, part_ref, wn1a, wn1b, bn1, wn2, bn2, wa, wb, ba,
               h_out, x_out, t_out):
    hnew, xnew = _node_common(h_ref, x_ref, part_ref, wn1a, wn1b, bn1, wn2, bn2)
    h_out[...] = hnew
    x_out[...] = xnew
    t_out[0, :, :D] = _dot(hnew, wa[...]) + ba[...]
    t_out[0, :, D:W] = xnew
    t_out[1, :, :D] = _dot(hnew, wb[...])
    t_out[1, :, D:W] = -xnew


def _node_last_body(h_ref, x_ref, part_ref, wn1a, wn1b, bn1, wn2, bn2, wlh, blh,
                    out_ref):
    hnew, _ = _node_common(h_ref, x_ref, part_ref, wn1a, wn1b, bn1, wn2, bn2)
    out_ref[...] = _dot(hnew, wlh[...]) + blh[...]


_node_in_specs = [
    pl.BlockSpec((BN, D), lambda i: (i, 0)),
    pl.BlockSpec((BN, XW), lambda i: (i, 0)),
    pl.BlockSpec((NC, BN, W), lambda i: (0, i, 0)),
    _wspec((D, D)),
    _wspec((D, D)),
    _wspec((1, D)),
    _wspec((D, D)),
    _wspec((1, D)),
]

_node = pl.pallas_call(
    _node_body,
    grid=(N // BN,),
    in_specs=_node_in_specs + [_wspec((D, D)), _wspec((D, D)), _wspec((1, D))],
    out_specs=[
        pl.BlockSpec((BN, D), lambda i: (i, 0)),
        pl.BlockSpec((BN, XW), lambda i: (i, 0)),
        pl.BlockSpec((2, BN, W), lambda i: (0, i, 0)),
    ],
    out_shape=[
        jax.ShapeDtypeStruct((N, D), jnp.float32),
        jax.ShapeDtypeStruct((N, XW), jnp.float32),
        jax.ShapeDtypeStruct((2, N, W), jnp.float32),
    ],
)

_node_last = pl.pallas_call(
    _node_last_body,
    grid=(N // BN,),
    in_specs=_node_in_specs + [_wspec((D, C)), _wspec((1, C))],
    out_specs=pl.BlockSpec((BN, C), lambda i: (i, 0)),
    out_shape=jax.ShapeDtypeStruct((N, C), jnp.float32),
)


def kernel(in_feat, coord_feat, edge_index, We1, be1, We2, be2, Wc1, bc1, Wc2,
           Wn1, bn1, Wn2, bn2, Wlh, blh):
    src = edge_index[0]
    dst = edge_index[1]
    src_i = src.reshape(NCHUNK, BC)
    dstg_i = (dst + N).reshape(NCHUNK, BC)
    dsts_i = dst.reshape(NCHUNK, BC)
    x = jnp.concatenate([coord_feat, jnp.zeros((N, XW - 3), jnp.float32)], axis=1)
    zrows = jnp.zeros((BC, W), jnp.float32)
    h = in_feat
    tab = _prep(h, x, We1[0, :D], We1[0, D:2 * D], be1[0][None])
    out = None
    for i in range(LAYERS):
        ein = _gather(tab.reshape(2 * N, W), src_i, dstg_i)
        msg = _edge(ein, We1[i, 2 * D][None], We2[i], be2[i][None],
                    Wc1[i], bc1[i][None], Wc2[i])
        part = _scatter(msg, dsts_i, zrows)
        if i < LAYERS - 1:
            h, x, tab = _node(h, x, part, Wn1[i, :D], Wn1[i, D:],
                              bn1[i][None], Wn2[i], bn2[i][None],
                              We1[i + 1, :D], We1[i + 1, D:2 * D],
                              be1[i + 1][None])
        else:
            out = _node_last(h, x, part, Wn1[i, :D], Wn1[i, D:],
                             bn1[i][None], Wn2[i], bn2[i][None], Wlh, blh[None])
    return out
